# Initial kernel scaffold; baseline (speedup 1.0000x reference)
import jax
import jax.numpy as jnp
from jax.experimental import pallas as pl


def kernel(hidden_states, router_w, w1, v1, w2):
    raise NotImplementedError("write your pallas kernel here")


# speedup vs baseline: 1.6318x; 1.6318x over previous
import functools

import jax
import jax.numpy as jnp
from jax.experimental import pallas as pl

E = 8
TOP_K = 2
H = 1024
F = 1024
T = 2048
LANES = 128
TBLK = 512


def _router_body(x_ref, rw_ref, comb_ref):
    x = x_ref[...]
    rw = rw_ref[...]
    logits = jax.lax.dot_general(
        x, rw, (((1,), (1,)), ((), ())), preferred_element_type=jnp.float32
    )
    lane = jax.lax.broadcasted_iota(jnp.int32, logits.shape, 1)
    neg = jnp.float32(-1e30)
    logits = jnp.where(lane < E, logits, neg)
    m = jnp.max(logits, axis=1, keepdims=True)
    ex = jnp.exp(logits - m)
    ex = jnp.where(lane < E, ex, 0.0)
    scores = ex / jnp.sum(ex, axis=1, keepdims=True)
    big = jnp.int32(LANES)
    m1 = jnp.max(scores, axis=1, keepdims=True)
    i1 = jnp.min(jnp.where(scores == m1, lane, big), axis=1, keepdims=True)
    sc2 = jnp.where(lane == i1, neg, scores)
    m2 = jnp.max(sc2, axis=1, keepdims=True)
    i2 = jnp.min(jnp.where(sc2 == m2, lane, big), axis=1, keepdims=True)
    denom = m1 + m2
    comb = jnp.where(lane == i1, m1 / denom, 0.0) + jnp.where(lane == i2, m2 / denom, 0.0)
    comb_ref[...] = comb


def _expert_body(x_ref, comb_ref, w1_ref, v1_ref, w2_ref, out_ref):
    e = pl.program_id(1)
    xb = x_ref[...].astype(jnp.bfloat16)
    w1b = w1_ref[0].astype(jnp.bfloat16)
    v1b = v1_ref[0].astype(jnp.bfloat16)
    w2b = w2_ref[0].astype(jnp.bfloat16)
    h1 = jax.lax.dot_general(
        xb, w1b, (((1,), (1,)), ((), ())), preferred_element_type=jnp.float32
    )
    h2 = jax.lax.dot_general(
        xb, v1b, (((1,), (1,)), ((), ())), preferred_element_type=jnp.float32
    )
    h = (h1 * jax.nn.sigmoid(h1) * h2).astype(jnp.bfloat16)
    y = jax.lax.dot_general(
        h, w2b, (((1,), (0,)), ((), ())), preferred_element_type=jnp.float32
    )
    lane = jax.lax.broadcasted_iota(jnp.int32, comb_ref.shape, 1)
    c = jnp.sum(jnp.where(lane == e, comb_ref[...], 0.0), axis=1, keepdims=True)
    contrib = c * y

    @pl.when(e == 0)
    def _():
        out_ref[...] = contrib

    @pl.when(e > 0)
    def _():
        out_ref[...] += contrib


@jax.jit
def kernel(hidden_states, router_w, w1, v1, w2):
    xf = hidden_states.reshape(T, H)
    rw = jnp.zeros((LANES, H), jnp.float32).at[:E].set(router_w)

    comb = pl.pallas_call(
        _router_body,
        grid=(1,),
        in_specs=[
            pl.BlockSpec((T, H), lambda i: (0, 0)),
            pl.BlockSpec((LANES, H), lambda i: (0, 0)),
        ],
        out_specs=pl.BlockSpec((T, LANES), lambda i: (0, 0)),
        out_shape=jax.ShapeDtypeStruct((T, LANES), jnp.float32),
    )(xf, rw)

    out = pl.pallas_call(
        _expert_body,
        grid=(T // TBLK, E),
        in_specs=[
            pl.BlockSpec((TBLK, H), lambda t, e: (t, 0)),
            pl.BlockSpec((TBLK, LANES), lambda t, e: (t, 0)),
            pl.BlockSpec((1, F, H), lambda t, e: (e, 0, 0)),
            pl.BlockSpec((1, F, H), lambda t, e: (e, 0, 0)),
            pl.BlockSpec((1, F, H), lambda t, e: (e, 0, 0)),
        ],
        out_specs=pl.BlockSpec((TBLK, H), lambda t, e: (t, 0)),
        out_shape=jax.ShapeDtypeStruct((T, H), jnp.float32),
    )(xf, comb, w1, v1, w2)

    return out.reshape(1, T, H)
